# baseline (speedup 1.0000x reference)
import functools

import jax
import jax.numpy as jnp
from jax import lax
from jax.experimental import pallas as pl
from jax.experimental.pallas import tpu as pltpu
from jax.experimental.pallas import tpu_sc as plsc

_CHUNK = 128


@functools.lru_cache(maxsize=None)
def _make_gather(V, D, B):
    info = plsc.get_sparse_core_info()
    NC, NS = info.num_cores, info.num_subcores
    NW = NC * NS
    n_chunks = B // _CHUNK
    C = n_chunks // NW
    mesh = plsc.VectorSubcoreMesh(core_axis_name="c", subcore_axis_name="s")

    @functools.partial(
        pl.kernel,
        mesh=mesh,
        out_type=jax.ShapeDtypeStruct((n_chunks, _CHUNK, D), jnp.float32),
        scratch_types=[
            pltpu.VMEM((C, _CHUNK), jnp.int32),
            pltpu.VMEM((C, _CHUNK, D), jnp.float32),
            pltpu.SemaphoreType.DMA,
        ],
        compiler_params=pltpu.CompilerParams(use_tc_tiling_on_sc=False),
    )
    def gather_kernel(table_hbm, idx_hbm, out_hbm, idx_v, rows_v, sem):
        wid = lax.axis_index("s") * NC + lax.axis_index("c")
        base = wid * C
        pltpu.sync_copy(idx_hbm.at[pl.ds(base, C)], idx_v)
        copies = [
            pltpu.async_copy(table_hbm.at[idx_v.at[j]], rows_v.at[j], sem)
            for j in range(C)
        ]
        for cp in copies:
            cp.wait()
        pltpu.sync_copy(rows_v, out_hbm.at[pl.ds(base, C)])

    return gather_kernel


def kernel(inputs, embedding):
    B = inputs.shape[0]
    V = embedding.shape[0]
    D = embedding.shape[-1]
    table = embedding.reshape(V, D)
    idx = inputs.astype(jnp.int32).reshape(B // _CHUNK, _CHUNK)
    out = _make_gather(V, D, B)(table, idx)
    return out.reshape(inputs.shape + (1, 1, D))


# speedup vs baseline: 1.8565x; 1.8565x over previous
import functools

import jax
import jax.numpy as jnp
from jax import lax
from jax.experimental import pallas as pl
from jax.experimental.pallas import tpu as pltpu
from jax.experimental.pallas import tpu_sc as plsc

_LANE = 128
_BLK = 128


@functools.lru_cache(maxsize=None)
def _make_gather(V, D, B):
    info = plsc.get_sparse_core_info()
    NC, NS = info.num_cores, info.num_subcores
    NW = NC * NS
    per_w = B // NW
    n_blk = per_w // _BLK
    mesh = plsc.VectorSubcoreMesh(core_axis_name="c", subcore_axis_name="s")

    @functools.partial(
        pl.kernel,
        mesh=mesh,
        out_type=jax.ShapeDtypeStruct((D, B), jnp.float32),
        scratch_types=[
            pltpu.VMEM((per_w,), jnp.int32),
            pltpu.VMEM((2, D, _LANE), jnp.float32),
            pltpu.VMEM((D, _BLK), jnp.float32),
            pltpu.SemaphoreType.DMA,
            pltpu.SemaphoreType.DMA,
        ],
        compiler_params=pltpu.CompilerParams(
            use_tc_tiling_on_sc=True, needs_layout_passes=False
        ),
    )
    def gather_kernel(table_hbm, idx_hbm, out_hbm, idx_v, blk_v, stage_v,
                      sem0, sem1):
        wid = lax.axis_index("s") * NC + lax.axis_index("c")
        b0 = wid * per_w
        pltpu.sync_copy(idx_hbm.at[pl.ds(b0, per_w)], idx_v)
        row_ids = lax.iota(jnp.int32, 16)
        sems = (sem0, sem1)

        def fire(r, slot):
            col = pl.multiple_of((r >> 7) * _LANE, _LANE)
            return pltpu.async_copy(
                table_hbm.at[:, pl.ds(col, _LANE)], blk_v.at[slot], sems[slot]
            )

        def extract(r, k, slot):
            lane_v = jnp.full((16,), r & (_LANE - 1), jnp.int32)
            k_v = jnp.full((16,), k, jnp.int32)
            for q in range(D // 16):
                f_v = row_ids + q * 16
                vals = plsc.load_gather(blk_v.at[slot], [f_v, lane_v])
                plsc.store_scatter(stage_v, [f_v, k_v], vals)

        def do_block(g, _):
            def do_chunk(ch, _):
                r_vec = idx_v[pl.ds(g * _BLK + ch * 16, 16)]
                cp = fire(r_vec[0], 0)
                for j in range(16):
                    if j < 15:
                        cp_next = fire(r_vec[j + 1], (j + 1) & 1)
                    cp.wait()
                    extract(r_vec[j], ch * 16 + j, j & 1)
                    if j < 15:
                        cp = cp_next
                return 0

            lax.fori_loop(0, _BLK // 16, do_chunk, 0)
            pltpu.sync_copy(stage_v, out_hbm.at[:, pl.ds(b0 + g * _BLK, _BLK)])
            return 0

        lax.fori_loop(0, n_blk, do_block, 0)

    return gather_kernel


def kernel(inputs, embedding):
    B = inputs.shape[0]
    V = embedding.shape[0]
    D = embedding.shape[-1]
    table_t = embedding.reshape(V, D).T
    idx = inputs.astype(jnp.int32)
    out_t = _make_gather(V, D, B)(table_t, idx)
    return out_t.T.reshape(inputs.shape + (1, 1, D))


# speedup vs baseline: 2.6607x; 1.4331x over previous
import functools

import jax
import jax.numpy as jnp
from jax import lax
from jax.experimental import pallas as pl
from jax.experimental.pallas import tpu as pltpu
from jax.experimental.pallas import tpu_sc as plsc

_LANE = 128
_BLK = 128
_NBUF = 8


@functools.lru_cache(maxsize=None)
def _make_gather(V, D, B):
    info = plsc.get_sparse_core_info()
    NC, NS = info.num_cores, info.num_subcores
    NW = NC * NS
    per_w = B // NW
    n_blk = per_w // _BLK
    mesh = plsc.VectorSubcoreMesh(core_axis_name="c", subcore_axis_name="s")

    @functools.partial(
        pl.kernel,
        mesh=mesh,
        out_type=jax.ShapeDtypeStruct((D, B), jnp.float32),
        scratch_types=[
            pltpu.VMEM((per_w,), jnp.int32),
            pltpu.VMEM((_NBUF, D, _LANE), jnp.float32),
            pltpu.VMEM((D, _BLK), jnp.float32),
        ] + [pltpu.SemaphoreType.DMA] * _NBUF,
        compiler_params=pltpu.CompilerParams(
            use_tc_tiling_on_sc=True, needs_layout_passes=False
        ),
    )
    def gather_kernel(table_hbm, idx_hbm, out_hbm, idx_v, blk_v, stage_v,
                      *sems):
        wid = lax.axis_index("s") * NC + lax.axis_index("c")
        b0 = wid * per_w
        pltpu.sync_copy(idx_hbm.at[pl.ds(b0, per_w)], idx_v)
        row_ids = lax.iota(jnp.int32, 16)

        def fire(r, slot):
            col = pl.multiple_of((r >> 7) * _LANE, _LANE)
            return pltpu.async_copy(
                table_hbm.at[:, pl.ds(col, _LANE)], blk_v.at[slot], sems[slot]
            )

        def extract(r, k, slot):
            lane_v = jnp.full((16,), r & (_LANE - 1), jnp.int32)
            k_v = jnp.full((16,), k, jnp.int32)
            for q in range(D // 16):
                f_v = row_ids + q * 16
                vals = plsc.load_gather(blk_v.at[slot], [f_v, lane_v])
                plsc.store_scatter(stage_v, [f_v, k_v], vals)

        def do_block(g, _):
            def do_chunk(ch, _):
                r_vec = idx_v[pl.ds(g * _BLK + ch * 16, 16)]
                cps = [fire(r_vec[j], j) for j in range(_NBUF)]
                for j in range(16):
                    cps[j % _NBUF].wait()
                    extract(r_vec[j], ch * 16 + j, j % _NBUF)
                    if j + _NBUF < 16:
                        cps[j % _NBUF] = fire(r_vec[j + _NBUF], j % _NBUF)
                return 0

            lax.fori_loop(0, _BLK // 16, do_chunk, 0)
            pltpu.sync_copy(stage_v, out_hbm.at[:, pl.ds(b0 + g * _BLK, _BLK)])
            return 0

        lax.fori_loop(0, n_blk, do_block, 0)

    return gather_kernel


def kernel(inputs, embedding):
    B = inputs.shape[0]
    V = embedding.shape[0]
    D = embedding.shape[-1]
    table_t = embedding.reshape(V, D).T
    idx = inputs.astype(jnp.int32)
    out_t = _make_gather(V, D, B)(table_t, idx)
    return out_t.T.reshape(inputs.shape + (1, 1, D))


# speedup vs baseline: 2.8153x; 1.0581x over previous
import functools

import jax
import jax.numpy as jnp
from jax import lax
from jax.experimental import pallas as pl
from jax.experimental.pallas import tpu as pltpu
from jax.experimental.pallas import tpu_sc as plsc

_LANE = 128
_BLK = 128
_NBUF = 8


@functools.lru_cache(maxsize=None)
def _make_gather(V, D, B):
    info = plsc.get_sparse_core_info()
    NC, NS = info.num_cores, info.num_subcores
    NW = NC * NS
    per_w = B // NW
    n_ch = per_w // 16
    flushes = _BLK // 16
    max_col = ((V // _LANE) - 1) * _LANE
    mesh = plsc.VectorSubcoreMesh(core_axis_name="c", subcore_axis_name="s")

    @functools.partial(
        pl.kernel,
        mesh=mesh,
        out_type=jax.ShapeDtypeStruct((D, B), jnp.float32),
        scratch_types=[
            pltpu.VMEM((per_w + 16,), jnp.int32),
            pltpu.VMEM((_NBUF, D, _LANE), jnp.float32),
            pltpu.VMEM((D, _BLK), jnp.float32),
        ] + [pltpu.SemaphoreType.DMA] * _NBUF,
        compiler_params=pltpu.CompilerParams(
            use_tc_tiling_on_sc=True, needs_layout_passes=False
        ),
    )
    def gather_kernel(table_hbm, idx_hbm, out_hbm, idx_v, blk_v, stage_v,
                      *sems):
        wid = lax.axis_index("s") * NC + lax.axis_index("c")
        b0 = wid * per_w
        pltpu.sync_copy(idx_hbm.at[pl.ds(b0, per_w)], idx_v.at[pl.ds(0, per_w)])
        row_ids = lax.iota(jnp.int32, 16)

        def fire(r, slot):
            col = jnp.clip((r >> 7) * _LANE, 0, max_col)
            col = pl.multiple_of(col, _LANE)
            return pltpu.async_copy(
                table_hbm.at[:, pl.ds(col, _LANE)], blk_v.at[slot], sems[slot]
            )

        def slot_wait(slot):
            pltpu.make_async_copy(
                table_hbm.at[:, pl.ds(0, _LANE)], blk_v.at[slot], sems[slot]
            ).wait()

        def extract(r, k, slot):
            lane_v = jnp.full((16,), r & (_LANE - 1), jnp.int32)
            k_v = jnp.full((16,), k, jnp.int32)
            for q in range(D // 16):
                f_v = row_ids + q * 16
                vals = plsc.load_gather(blk_v.at[slot], [f_v, lane_v])
                plsc.store_scatter(stage_v, [f_v, k_v], vals)

        r0 = idx_v[pl.ds(0, 16)]
        for j in range(_NBUF):
            fire(r0[j], j)

        def do_chunk(ch, _):
            base = ch * 16
            r_cur = idx_v[pl.ds(base, 16)]
            r_nxt = idx_v[pl.ds(base + 16, 16)]
            kc = (ch % flushes) * 16
            for j in range(16):
                slot = j % _NBUF
                slot_wait(slot)
                extract(r_cur[j], kc + j, slot)
                if j < _NBUF:
                    fire(r_cur[j + _NBUF], slot)
                else:
                    fire(r_nxt[j - _NBUF], slot)

            @pl.when(ch % flushes == flushes - 1)
            def _flush():
                start = pl.multiple_of(b0 + (ch + 1) * 16 - _BLK, _BLK)
                pltpu.sync_copy(stage_v, out_hbm.at[:, pl.ds(start, _BLK)])

            return 0

        lax.fori_loop(0, n_ch, do_chunk, 0)
        for j in range(_NBUF):
            slot_wait(j)

    return gather_kernel


def kernel(inputs, embedding):
    B = inputs.shape[0]
    V = embedding.shape[0]
    D = embedding.shape[-1]
    table_t = embedding.reshape(V, D).T
    idx = inputs.astype(jnp.int32)
    out_t = _make_gather(V, D, B)(table_t, idx)
    return out_t.T.reshape(inputs.shape + (1, 1, D))


# speedup vs baseline: 2.9039x; 1.0315x over previous
import functools

import jax
import jax.numpy as jnp
from jax import lax
from jax.experimental import pallas as pl
from jax.experimental.pallas import tpu as pltpu
from jax.experimental.pallas import tpu_sc as plsc

_LANE = 128
_BLK = 128
_NBUF = 8


@functools.lru_cache(maxsize=None)
def _make_gather(V, D, B):
    info = plsc.get_sparse_core_info()
    NC, NS = info.num_cores, info.num_subcores
    NW = NC * NS
    per_w = B // NW
    n_ch = per_w // 16
    flushes = _BLK // 16
    max_col = ((V - 1) >> 7) * _LANE
    mesh = plsc.VectorSubcoreMesh(core_axis_name="c", subcore_axis_name="s")

    @functools.partial(
        pl.kernel,
        mesh=mesh,
        out_type=jax.ShapeDtypeStruct((D, B), jnp.float32),
        scratch_types=[
            pltpu.VMEM((per_w + 16,), jnp.int32),
            pltpu.VMEM((_NBUF, D, _LANE), jnp.float32),
            pltpu.VMEM((D, _BLK), jnp.float32),
        ] + [pltpu.SemaphoreType.DMA] * _NBUF,
        compiler_params=pltpu.CompilerParams(
            use_tc_tiling_on_sc=True, needs_layout_passes=False
        ),
    )
    def gather_kernel(table_hbm, idx_hbm, out_hbm, idx_v, blk_v, stage_v,
                      *sems):
        wid = lax.axis_index("s") * NC + lax.axis_index("c")
        b0 = wid * per_w
        pltpu.sync_copy(idx_hbm.at[pl.ds(b0, per_w)], idx_v.at[pl.ds(0, per_w)])
        row_ids = lax.iota(jnp.int32, 16)

        def fire(r, slot):
            col = jnp.clip((r >> 7) * _LANE, 0, max_col)
            col = pl.multiple_of(col, _LANE)
            return pltpu.async_copy(
                table_hbm.at[:, pl.ds(col, _LANE)], blk_v.at[slot], sems[slot]
            )

        def slot_wait(slot):
            pltpu.make_async_copy(
                table_hbm.at[:, pl.ds(0, _LANE)], blk_v.at[slot], sems[slot]
            ).wait()

        def extract(r, k, slot):
            lane_v = jnp.full((16,), r & (_LANE - 1), jnp.int32)
            k_v = jnp.full((16,), k, jnp.int32)
            for q in range(D // 16):
                f_v = row_ids + q * 16
                vals = plsc.load_gather(blk_v.at[slot], [f_v, lane_v])
                plsc.store_scatter(stage_v, [f_v, k_v], vals)

        r0 = idx_v[pl.ds(0, 16)]
        for j in range(_NBUF):
            fire(r0[j], j)

        def do_chunk(ch, _):
            base = ch * 16
            r_cur = idx_v[pl.ds(base, 16)]
            r_nxt = idx_v[pl.ds(base + 16, 16)]
            kc = (ch % flushes) * 16
            for j in range(16):
                slot = j % _NBUF
                slot_wait(slot)
                extract(r_cur[j], kc + j, slot)
                if j < _NBUF:
                    fire(r_cur[j + _NBUF], slot)
                else:
                    fire(r_nxt[j - _NBUF], slot)

            @pl.when(ch % flushes == flushes - 1)
            def _flush():
                start = pl.multiple_of(b0 + (ch + 1) * 16 - _BLK, _BLK)
                pltpu.sync_copy(stage_v, out_hbm.at[:, pl.ds(start, _BLK)])

            return 0

        lax.fori_loop(0, n_ch, do_chunk, 0)
        for j in range(_NBUF):
            slot_wait(j)

    return gather_kernel


def kernel(inputs, embedding):
    B = inputs.shape[0]
    V = embedding.shape[0]
    D = embedding.shape[-1]
    table_t = embedding.reshape(V, D).T
    idx = inputs.astype(jnp.int32)
    out_t = _make_gather(V, D, B)(table_t, idx)
    return out_t.T.reshape(inputs.shape + (1, 1, D))
